# Initial kernel scaffold; baseline (speedup 1.0000x reference)
import functools
import numpy as np
import jax
import jax.numpy as jnp
from jax import lax
from jax.experimental import pallas as pl
from jax.experimental.pallas import tpu as pltpu
from jax.experimental.pallas import tpu_sc as plsc

_A, _S, _D, _Q, _H = 100000, 16, 32, 128, 1
_B = 16384
_ROW = _S * _D

_NC, _NS = 2, 16
_NW = _NC * _NS
_BPW = _B // _NW
_CH = 64
_NCHUNK = _BPW // _CH


def _sc_gather_body(table_hbm, idx_hbm, out_hbm, idx_v, rows_v, sems):
    wid = lax.axis_index("s") * _NC + lax.axis_index("c")
    base = wid * _BPW
    pltpu.sync_copy(idx_hbm.at[pl.ds(base, _BPW)], idx_v)
    cps = [
        pltpu.async_copy(
            table_hbm.at[idx_v.at[pl.ds(c * _CH, _CH)]],
            rows_v.at[c % 2],
            sems.at[c % 2],
        )
        for c in range(2)
    ] + [None] * (_NCHUNK - 2)
    for c in range(_NCHUNK):
        cps[c].wait()
        if c + 2 < _NCHUNK:
            cps[c + 2] = pltpu.async_copy(
                table_hbm.at[idx_v.at[pl.ds((c + 2) * _CH, _CH)]],
                rows_v.at[c % 2],
                sems.at[c % 2],
            )
        pltpu.sync_copy(rows_v.at[c % 2], out_hbm.at[pl.ds(base + c * _CH, _CH)])


_sc_gather = functools.partial(
    pl.kernel,
    out_type=jax.ShapeDtypeStruct((_B, _ROW), jnp.float32),
    mesh=plsc.VectorSubcoreMesh(core_axis_name="c", subcore_axis_name="s"),
    scratch_types=[
        pltpu.VMEM((_BPW,), jnp.int32),
        pltpu.VMEM((2, _CH, _ROW), jnp.float32),
        pltpu.SemaphoreType.DMA((2,)),
    ],
)(_sc_gather_body)


_d = np.arange(_ROW)
_TILE = (_d % _D == np.arange(_D)[:, None]).astype(np.float32)
_EXPAND = (_d // _D == np.arange(_S)[:, None]).astype(np.float32)
_SEG = _EXPAND.T.copy()
_REP = _TILE.T.copy()

_INV_SQRT_D = np.float32(1.0 / np.sqrt(_D))
_INV_SQRT_2 = np.float32(1.0 / np.sqrt(2.0))


def _dense_body(q_ref, g_ref, wq_ref, bq_ref, wf_ref, bf_ref, lng_ref, lnb_ref,
                tile_ref, seg_ref, exp_ref, rep_ref, out_ref):
    q = q_ref[...]
    g = g_ref[...]
    pq = jnp.dot(q, wq_ref[...], preferred_element_type=jnp.float32) + bq_ref[...]
    qn = pq / jnp.maximum(
        jnp.sqrt(jnp.sum(pq * pq, axis=1, keepdims=True)), 1e-12)
    qt = jnp.dot(qn, tile_ref[...], preferred_element_type=jnp.float32)
    dots = jnp.dot(g * qt, seg_ref[...], preferred_element_type=jnp.float32)
    ssq = jnp.dot(g * g, seg_ref[...], preferred_element_type=jnp.float32)
    scores = dots / jnp.maximum(jnp.sqrt(ssq), 1e-12)
    scaled = scores * _INV_SQRT_D
    mx = jnp.max(scaled, axis=1, keepdims=True)
    e = jnp.exp(scaled - mx)
    w = e / jnp.sum(e, axis=1, keepdims=True)
    wt = jnp.dot(w, exp_ref[...], preferred_element_type=jnp.float32)
    rv = jnp.dot(g * wt, rep_ref[...], preferred_element_type=jnp.float32)
    hdn = jnp.dot(rv, wf_ref[...], preferred_element_type=jnp.float32) + bf_ref[...]
    hdn = hdn * 0.5 * (1.0 + lax.erf(hdn * _INV_SQRT_2))
    mu = jnp.mean(hdn, axis=1, keepdims=True)
    ctr = hdn - mu
    var = jnp.mean(ctr * ctr, axis=1, keepdims=True)
    out_ref[...] = ctr * jax.lax.rsqrt(var + 1e-5) * lng_ref[...] + lnb_ref[...]


def _dense(gathered, queries, Wq, bq, Wf, bf, ln_g, ln_b, *, bsz=2048,
           interpret=False):
    nblk = _B // bsz
    full = lambda i: (0, 0)
    blk = lambda i: (i, 0)
    return pl.pallas_call(
        _dense_body,
        grid=(nblk,),
        in_specs=[
            pl.BlockSpec((bsz, _Q), blk),
            pl.BlockSpec((bsz, _ROW), blk),
            pl.BlockSpec((_Q, _D), full),
            pl.BlockSpec((1, _D), full),
            pl.BlockSpec((_D, _Q), full),
            pl.BlockSpec((1, _Q), full),
            pl.BlockSpec((1, _Q), full),
            pl.BlockSpec((1, _Q), full),
            pl.BlockSpec((_D, _ROW), full),
            pl.BlockSpec((_ROW, _S), full),
            pl.BlockSpec((_S, _ROW), full),
            pl.BlockSpec((_ROW, _D), full),
        ],
        out_specs=pl.BlockSpec((bsz, _Q), blk),
        out_shape=jax.ShapeDtypeStruct((_B, _Q), jnp.float32),
        interpret=interpret,
    )(queries, gathered, Wq, bq.reshape(1, _D), Wf, bf.reshape(1, _Q),
      ln_g.reshape(1, _Q), ln_b.reshape(1, _Q),
      jnp.asarray(_TILE), jnp.asarray(_SEG), jnp.asarray(_EXPAND),
      jnp.asarray(_REP))


def kernel(queries, memory, Wq, bq, Wf, bf, ln_g, ln_b, agent_indices):
    table = memory.reshape(_A, _ROW)
    gathered = _sc_gather(table, agent_indices.astype(jnp.int32))
    return _dense(gathered, queries, Wq, bq, Wf, bf, ln_g, ln_b)


# speedup vs baseline: 9.2230x; 9.2230x over previous
import functools
import numpy as np
import jax
import jax.numpy as jnp
from jax import lax
from jax.experimental import pallas as pl
from jax.experimental.pallas import tpu as pltpu
from jax.experimental.pallas import tpu_sc as plsc

_A, _S, _D, _Q, _H = 100000, 16, 32, 128, 1
_B = 16384
_ROW = _S * _D

_NC, _NS = 2, 16
_NW = _NC * _NS
_BPW = _B // _NW
_CH = 64
_NCHUNK = _BPW // _CH


def _sc_gather_body(table_hbm, idx_hbm, out_hbm, idx_v, rows_v, sems):
    wid = lax.axis_index("s") * _NC + lax.axis_index("c")
    base = wid * _BPW
    pltpu.sync_copy(idx_hbm.at[pl.ds(base, _BPW)], idx_v)
    cps = [
        pltpu.async_copy(
            table_hbm.at[idx_v.at[pl.ds(c * _CH, _CH)]],
            rows_v.at[c % 2],
            sems.at[c % 2],
        )
        for c in range(2)
    ] + [None] * (_NCHUNK - 2)
    for c in range(_NCHUNK):
        cps[c].wait()
        pltpu.sync_copy(rows_v.at[c % 2], out_hbm.at[pl.ds(base + c * _CH, _CH)])
        if c + 2 < _NCHUNK:
            cps[c + 2] = pltpu.async_copy(
                table_hbm.at[idx_v.at[pl.ds((c + 2) * _CH, _CH)]],
                rows_v.at[c % 2],
                sems.at[c % 2],
            )


@functools.lru_cache(maxsize=None)
def _make_sc_gather():
    return pl.kernel(
        _sc_gather_body,
        out_type=jax.ShapeDtypeStruct((_B, _ROW), jnp.float32),
        mesh=plsc.VectorSubcoreMesh(core_axis_name="c", subcore_axis_name="s"),
        scratch_types=[
            pltpu.VMEM((_BPW,), jnp.int32),
            pltpu.VMEM((2, _CH, _ROW), jnp.float32),
            pltpu.SemaphoreType.DMA((2,)),
        ],
    )


_d = np.arange(_ROW)
_TILE = (_d % _D == np.arange(_D)[:, None]).astype(np.float32)
_EXPAND = (_d // _D == np.arange(_S)[:, None]).astype(np.float32)
_SEG = _EXPAND.T.copy()
_REP = _TILE.T.copy()

_INV_SQRT_D = np.float32(1.0 / np.sqrt(_D))
_INV_SQRT_2 = np.float32(1.0 / np.sqrt(2.0))


def _dense_body(q_ref, g_ref, wq_ref, bq_ref, wf_ref, bf_ref, lng_ref, lnb_ref,
                tile_ref, seg_ref, exp_ref, rep_ref, out_ref):
    q = q_ref[...]
    g = g_ref[...]
    pq = jnp.dot(q, wq_ref[...], preferred_element_type=jnp.float32) + bq_ref[...]
    qn = pq / jnp.maximum(
        jnp.sqrt(jnp.sum(pq * pq, axis=1, keepdims=True)), 1e-12)
    qt = jnp.dot(qn, tile_ref[...], preferred_element_type=jnp.float32)
    dots = jnp.dot(g * qt, seg_ref[...], preferred_element_type=jnp.float32)
    ssq = jnp.dot(g * g, seg_ref[...], preferred_element_type=jnp.float32)
    scores = dots / jnp.maximum(jnp.sqrt(ssq), 1e-12)
    scaled = scores * _INV_SQRT_D
    mx = jnp.max(scaled, axis=1, keepdims=True)
    e = jnp.exp(scaled - mx)
    w = e / jnp.sum(e, axis=1, keepdims=True)
    wt = jnp.dot(w, exp_ref[...], preferred_element_type=jnp.float32)
    rv = jnp.dot(g * wt, rep_ref[...], preferred_element_type=jnp.float32)
    hdn = jnp.dot(rv, wf_ref[...], preferred_element_type=jnp.float32) + bf_ref[...]
    hdn = hdn * 0.5 * (1.0 + lax.erf(hdn * _INV_SQRT_2))
    mu = jnp.mean(hdn, axis=1, keepdims=True)
    ctr = hdn - mu
    var = jnp.mean(ctr * ctr, axis=1, keepdims=True)
    out_ref[...] = ctr * jax.lax.rsqrt(var + 1e-5) * lng_ref[...] + lnb_ref[...]


def _dense(gathered, queries, Wq, bq, Wf, bf, ln_g, ln_b, *, bsz=2048,
           interpret=False):
    nblk = _B // bsz
    full = lambda i: (0, 0)
    blk = lambda i: (i, 0)
    return pl.pallas_call(
        _dense_body,
        grid=(nblk,),
        in_specs=[
            pl.BlockSpec((bsz, _Q), blk),
            pl.BlockSpec((bsz, _ROW), blk),
            pl.BlockSpec((_Q, _D), full),
            pl.BlockSpec((1, _D), full),
            pl.BlockSpec((_D, _Q), full),
            pl.BlockSpec((1, _Q), full),
            pl.BlockSpec((1, _Q), full),
            pl.BlockSpec((1, _Q), full),
            pl.BlockSpec((_D, _ROW), full),
            pl.BlockSpec((_ROW, _S), full),
            pl.BlockSpec((_S, _ROW), full),
            pl.BlockSpec((_ROW, _D), full),
        ],
        out_specs=pl.BlockSpec((bsz, _Q), blk),
        out_shape=jax.ShapeDtypeStruct((_B, _Q), jnp.float32),
        interpret=interpret,
    )(queries, gathered, Wq, bq.reshape(1, _D), Wf, bf.reshape(1, _Q),
      ln_g.reshape(1, _Q), ln_b.reshape(1, _Q),
      jnp.asarray(_TILE), jnp.asarray(_SEG), jnp.asarray(_EXPAND),
      jnp.asarray(_REP))


def kernel(queries, memory, Wq, bq, Wf, bf, ln_g, ln_b, agent_indices):
    table = memory.reshape(_A, _ROW)
    gathered = _make_sc_gather()(table, agent_indices.astype(jnp.int32))
    return _dense(gathered, queries, Wq, bq, Wf, bf, ln_g, ln_b)


# speedup vs baseline: 10.9057x; 1.1824x over previous
import functools
import numpy as np
import jax
import jax.numpy as jnp
from jax import lax
from jax.experimental import pallas as pl
from jax.experimental.pallas import tpu as pltpu
from jax.experimental.pallas import tpu_sc as plsc

_A, _S, _D, _Q, _H = 100000, 16, 32, 128, 1
_B = 16384
_ROW = _S * _D

_NC, _NS = 2, 16
_NW = _NC * _NS
_FPW = _ROW // _NW
_HB = _B // 2


def _sc_scan_body(mem_t, idx_hbm, out_hbm, tab_v, idx_v, col_v):
    wid = lax.axis_index("s") * _NC + lax.axis_index("c")
    pltpu.sync_copy(idx_hbm, idx_v)
    for k in range(_FPW):
        j = wid * _FPW + k
        s = j // _D
        d = lax.rem(j, _D)
        pltpu.sync_copy(mem_t.at[s, d], tab_v)
        for half in range(2):

            def body(i, carry, half=half):
                for u in range(8):
                    off = i * 128 + u * 16
                    iv = idx_v[pl.ds(half * _HB + off, 16)]
                    col_v[pl.ds(off, 16)] = plsc.load_gather(tab_v, [iv])
                return carry

            lax.fori_loop(0, _HB // 128, body, 0)
            pltpu.sync_copy(col_v, out_hbm.at[j, pl.ds(half * _HB, _HB)])


@functools.lru_cache(maxsize=None)
def _make_sc_scan():
    return pl.kernel(
        _sc_scan_body,
        out_type=jax.ShapeDtypeStruct((_ROW, _B), jnp.float32),
        mesh=plsc.VectorSubcoreMesh(core_axis_name="c", subcore_axis_name="s"),
        scratch_types=[
            pltpu.VMEM((_A,), jnp.float32),
            pltpu.VMEM((_B,), jnp.int32),
            pltpu.VMEM((_HB,), jnp.float32),
        ],
        compiler_params=pltpu.CompilerParams(needs_layout_passes=False),
    )


_d = np.arange(_ROW)
_TILE = (_d % _D == np.arange(_D)[:, None]).astype(np.float32)
_EXPAND = (_d // _D == np.arange(_S)[:, None]).astype(np.float32)
_SEG = _EXPAND.T.copy()
_REP = _TILE.T.copy()

_INV_SQRT_D = np.float32(1.0 / np.sqrt(_D))
_INV_SQRT_2 = np.float32(1.0 / np.sqrt(2.0))


def _dense_body(q_ref, g_ref, wq_ref, bq_ref, wf_ref, bf_ref, lng_ref, lnb_ref,
                tile_ref, seg_ref, exp_ref, rep_ref, out_ref):
    q = q_ref[...]
    gt = g_ref[...]
    pq = jnp.dot(q, wq_ref[...], preferred_element_type=jnp.float32) + bq_ref[...]
    qn = pq / jnp.maximum(
        jnp.sqrt(jnp.sum(pq * pq, axis=1, keepdims=True)), 1e-12)
    qnt = jnp.transpose(qn, (1, 0))
    qtt = jnp.dot(rep_ref[...], qnt, preferred_element_type=jnp.float32)
    dots = jnp.dot(exp_ref[...], gt * qtt, preferred_element_type=jnp.float32)
    ssq = jnp.dot(exp_ref[...], gt * gt, preferred_element_type=jnp.float32)
    scores = dots / jnp.maximum(jnp.sqrt(ssq), 1e-12)
    scaled = scores * _INV_SQRT_D
    mx = jnp.max(scaled, axis=0, keepdims=True)
    e = jnp.exp(scaled - mx)
    w = e / jnp.sum(e, axis=0, keepdims=True)
    wt = jnp.dot(seg_ref[...], w, preferred_element_type=jnp.float32)
    rvt = jnp.dot(tile_ref[...], gt * wt, preferred_element_type=jnp.float32)
    rv = jnp.transpose(rvt, (1, 0))
    hdn = jnp.dot(rv, wf_ref[...], preferred_element_type=jnp.float32) + bf_ref[...]
    hdn = hdn * 0.5 * (1.0 + lax.erf(hdn * _INV_SQRT_2))
    mu = jnp.mean(hdn, axis=1, keepdims=True)
    ctr = hdn - mu
    var = jnp.mean(ctr * ctr, axis=1, keepdims=True)
    out_ref[...] = ctr * jax.lax.rsqrt(var + 1e-5) * lng_ref[...] + lnb_ref[...]


def _dense(gathered_t, queries, Wq, bq, Wf, bf, ln_g, ln_b, *, bsz=2048,
           interpret=False, nrows=_B):
    nblk = nrows // bsz
    full = lambda i: (0, 0)
    blk = lambda i: (i, 0)
    blkT = lambda i: (0, i)
    return pl.pallas_call(
        _dense_body,
        grid=(nblk,),
        in_specs=[
            pl.BlockSpec((bsz, _Q), blk),
            pl.BlockSpec((_ROW, bsz), blkT),
            pl.BlockSpec((_Q, _D), full),
            pl.BlockSpec((1, _D), full),
            pl.BlockSpec((_D, _Q), full),
            pl.BlockSpec((1, _Q), full),
            pl.BlockSpec((1, _Q), full),
            pl.BlockSpec((1, _Q), full),
            pl.BlockSpec((_D, _ROW), full),
            pl.BlockSpec((_ROW, _S), full),
            pl.BlockSpec((_S, _ROW), full),
            pl.BlockSpec((_ROW, _D), full),
        ],
        out_specs=pl.BlockSpec((bsz, _Q), blk),
        out_shape=jax.ShapeDtypeStruct((nrows, _Q), jnp.float32),
        interpret=interpret,
    )(queries, gathered_t, Wq, bq.reshape(1, _D), Wf, bf.reshape(1, _Q),
      ln_g.reshape(1, _Q), ln_b.reshape(1, _Q),
      jnp.asarray(_TILE), jnp.asarray(_SEG), jnp.asarray(_EXPAND),
      jnp.asarray(_REP))


def kernel(queries, memory, Wq, bq, Wf, bf, ln_g, ln_b, agent_indices):
    mem_t = jnp.transpose(memory, (1, 2, 0))
    gathered_t = _make_sc_scan()(mem_t, agent_indices.astype(jnp.int32))
    return _dense(gathered_t, queries, Wq, bq, Wf, bf, ln_g, ln_b)
